# baseline (speedup 1.0000x reference)
import functools

import jax
import jax.numpy as jnp
from jax import lax
from jax.experimental import pallas as pl
from jax.experimental.pallas import tpu as pltpu
from jax.experimental.pallas import tpu_sc as plsc

_ROWS = 262144
_DIM = 128
_NC = 2
_NS = 16
_NW = _NC * _NS
_ROWS_W = _ROWS // _NW
_CHUNK = 256
_NBUF = 2
_NITER = _ROWS_W // _CHUNK
_NGROUPS = _NITER // _NBUF


def _in_slice(in_hbm, row):
    return in_hbm.at[pl.ds(row, _CHUNK), :]


def _sc_copy_body(in_hbm, out_hbm, buf, in_sems, out_sems):
    wid = lax.axis_index("s") * _NC + lax.axis_index("c")
    base = wid * _ROWS_W

    for b in range(_NBUF):
        pltpu.make_async_copy(
            _in_slice(in_hbm, base + b * _CHUNK), buf.at[b], in_sems.at[b]
        ).start()

    def group(g, carry):
        for b in range(_NBUF):
            row = base + (g * _NBUF + b) * _CHUNK
            pltpu.make_async_copy(
                _in_slice(in_hbm, row), buf.at[b], in_sems.at[b]
            ).wait()
            out_cp = pltpu.make_async_copy(
                buf.at[b], out_hbm.at[pl.ds(row, _CHUNK), :], out_sems.at[b]
            )
            out_cp.start()
            out_cp.wait()
            nrow = row + _NBUF * _CHUNK
            pltpu.make_async_copy(
                _in_slice(in_hbm, nrow), buf.at[b], in_sems.at[b]
            ).start()
        return carry

    lax.fori_loop(0, _NGROUPS - 1, group, 0)

    for b in range(_NBUF):
        row = base + ((_NGROUPS - 1) * _NBUF + b) * _CHUNK
        pltpu.make_async_copy(
            _in_slice(in_hbm, row), buf.at[b], in_sems.at[b]
        ).wait()
        out_cp = pltpu.make_async_copy(
            buf.at[b], out_hbm.at[pl.ds(row, _CHUNK), :], out_sems.at[b]
        )
        out_cp.start()
        out_cp.wait()


_sc_copy = functools.partial(
    pl.kernel,
    out_type=jax.ShapeDtypeStruct((_ROWS, _DIM), jnp.float32),
    mesh=plsc.VectorSubcoreMesh(core_axis_name="c", subcore_axis_name="s"),
    scratch_types=[
        pltpu.VMEM((_NBUF, _CHUNK, _DIM), jnp.float32),
        pltpu.SemaphoreType.DMA((_NBUF,)),
        pltpu.SemaphoreType.DMA((_NBUF,)),
    ],
)(_sc_copy_body)


def kernel(queue):
    return _sc_copy(queue)
